# baseline (speedup 1.0000x reference)
import functools

import jax
import jax.numpy as jnp
from jax import lax
from jax.experimental import pallas as pl
from jax.experimental.pallas import tpu as pltpu
from jax.experimental.pallas import tpu_sc as plsc

_N, _C, _D, _H, _W = 4, 128, 2, 200, 176
_NNZ = 60000
_DHW = _D * _H * _W
_NDHW = _N * _DHW
_NWORK = 32
_PER_W = 1872
_LAST_W = _NNZ - 31 * _PER_W
_NCH = 16
_CH = 128
_SPARE = _NWORK * _CH
_GROWS = _NDHW + _SPARE
_MASKLEN = 2 * _GROWS
_MZ_PER_TILE = _GROWS // 16
_ZB = 2976


def _sc_body(features_hbm, indices_hbm, g_hbm, mask_hbm,
             idx_v, fbuf, lin_v, mlin_v, rowid_v, ones_v, zbuf):
    core = lax.axis_index("c")
    sub = lax.axis_index("s")
    wid = sub * 2 + core
    v0 = wid * _PER_W
    nw = jnp.where(wid == _NWORK - 1, _LAST_W, _PER_W)

    zeros16 = jnp.zeros((16,), jnp.int32)

    def fill(i, _):
        zbuf[pl.ds(i * 16, 16)] = zeros16
        return 0
    lax.fori_loop(0, _ZB // 16, fill, 0)

    def fill1(i, _):
        ones_v[pl.ds(i * 16, 16)] = zeros16 + 1
        return 0
    lax.fori_loop(0, _CH // 16, fill1, 0)

    mbase = core * _GROWS + sub * _MZ_PER_TILE
    for k in range(_MZ_PER_TILE // _ZB):
        pltpu.sync_copy(zbuf, mask_hbm.at[pl.ds(mbase + k * _ZB, _ZB)])
    plsc.subcore_barrier()

    pltpu.sync_copy(indices_hbm.at[pl.ds(v0 * 4, _LAST_W * 4)],
                    idx_v.at[pl.ds(0, _LAST_W * 4)])

    lane = lax.iota(jnp.int32, 16)

    def chunk(j, _):
        for s in range(_CH // 16):
            lid = j * _CH + s * 16 + lane
            valid = lid < nw
            rid = jnp.where(valid, lid, 0) * 4
            bv = plsc.load_gather(idx_v, [rid])
            zv = plsc.load_gather(idx_v, [rid + 1])
            yv = plsc.load_gather(idx_v, [rid + 2])
            xv = plsc.load_gather(idx_v, [rid + 3])
            lin = ((bv * _D + zv) * _H + yv) * _W + xv
            lin = jnp.where(valid, lin, _NDHW + wid * _CH + s * 16 + lane)
            lin_v[pl.ds(s * 16, 16)] = lin
            mlin_v[pl.ds(s * 16, 16)] = core * _GROWS + lin
            rowid_v[pl.ds(s * 16, 16)] = jnp.where(valid, v0 + lid, 0)
        pltpu.sync_copy(features_hbm.at[rowid_v], fbuf)
        pltpu.sync_copy(fbuf, g_hbm.at[lin_v])
        pltpu.sync_copy(ones_v, mask_hbm.at[mlin_v])
        return 0

    lax.fori_loop(0, _NCH, chunk, 0)


@functools.partial(
    pl.kernel,
    out_type=(
        jax.ShapeDtypeStruct((_GROWS, _C), jnp.float32),
        jax.ShapeDtypeStruct((_MASKLEN,), jnp.int32),
    ),
    mesh=plsc.VectorSubcoreMesh(core_axis_name="c", subcore_axis_name="s"),
    compiler_params=pltpu.CompilerParams(needs_layout_passes=False),
    scratch_types=[
        pltpu.VMEM((2048 * 4,), jnp.int32),
        pltpu.VMEM((_CH, _C), jnp.float32),
        pltpu.VMEM((_CH,), jnp.int32),
        pltpu.VMEM((_CH,), jnp.int32),
        pltpu.VMEM((_CH,), jnp.int32),
        pltpu.VMEM((_CH,), jnp.int32),
        pltpu.VMEM((_ZB,), jnp.int32),
    ],
)
def _sc_scatter(features_hbm, indices_hbm, g_hbm, mask_hbm, *scratch):
    _sc_body(features_hbm, indices_hbm, g_hbm, mask_hbm, *scratch)


def _tc_body(m_ref, g_ref, o_ref):
    m = m_ref[0, 0, 0, :] | m_ref[1, 0, 0, :]
    t = jnp.transpose(g_ref[...])
    o_ref[0] = jnp.where((m != 0)[None, :], t, jnp.float32(0.0))


def kernel(features, indices):
    g, mask = _sc_scatter(features, indices.reshape(-1))
    mask4 = mask.reshape(2, _GROWS // _CH, 1, _CH)
    out2 = pl.pallas_call(
        _tc_body,
        grid=(_N, _DHW // _CH),
        in_specs=[
            pl.BlockSpec((2, 1, 1, _CH), lambda n, j: (0, n * (_DHW // _CH) + j, 0, 0)),
            pl.BlockSpec((_CH, _C), lambda n, j: (n * (_DHW // _CH) + j, 0)),
        ],
        out_specs=pl.BlockSpec((1, _C, _CH), lambda n, j: (n, 0, j)),
        out_shape=jax.ShapeDtypeStruct((_N, _C, _DHW), jnp.float32),
    )(mask4, g)
    return out2.reshape(_N, _C * _D, _H, _W)


# speedup vs baseline: 2.4619x; 2.4619x over previous
import functools

import jax
import jax.numpy as jnp
from jax import lax
from jax.experimental import pallas as pl
from jax.experimental.pallas import tpu as pltpu
from jax.experimental.pallas import tpu_sc as plsc

_N, _C, _D, _H, _W = 4, 128, 2, 200, 176
_NNZ = 60000
_NHW = _N * _H * _W
_NWORK = 32
_PER_W = 1872
_LAST_W = _NNZ - 31 * _PER_W
_NCH = 16
_CH = 128
_RING = 4
_SPARE = _NWORK * _CH
_GROWS = 2 * _NHW + _SPARE
_MASKLEN = 2 * _GROWS
_MZ_PER_TILE = _GROWS // 16
_ZB = 2976
_CB = 256


def _sc_body(features_hbm, indices_hbm, g_hbm, mask_hbm,
             idx_v, fbuf, lin_v, mlin_v, rowid_v, ones_v, zbuf,
             zsem, msem, gsems, ssems):
    core = lax.axis_index("c")
    sub = lax.axis_index("s")
    wid = sub * 2 + core
    v0 = wid * _PER_W
    nw = jnp.where(wid == _NWORK - 1, _LAST_W, _PER_W)

    zeros16 = jnp.zeros((16,), jnp.float32)

    def fill(i, _):
        zbuf[pl.ds(i * 16, 16)] = zeros16
        return 0
    lax.fori_loop(0, _ZB // 16, fill, 0)

    def fill1(i, _):
        ones_v[pl.ds(i * 16, 16)] = zeros16 + 1.0
        return 0
    lax.fori_loop(0, _CH // 16, fill1, 0)

    zd = []
    mbase = core * _GROWS + sub * _MZ_PER_TILE
    for k in range(_MZ_PER_TILE // _ZB):
        zd.append(pltpu.async_copy(
            zbuf, mask_hbm.at[pl.ds(mbase + k * _ZB, _ZB)], zsem))

    pltpu.sync_copy(indices_hbm.at[pl.ds(v0 * 4, _LAST_W * 4)],
                    idx_v.at[pl.ds(0, _LAST_W * 4)])

    lane = lax.iota(jnp.int32, 16)

    def chunk(j, _):
        for s in range(_CH // 16):
            lid = j * _CH + s * 16 + lane
            valid = lid < nw
            rid = jnp.where(valid, lid, 0) * 4
            bv = plsc.load_gather(idx_v, [rid])
            zv = plsc.load_gather(idx_v, [rid + 1])
            yv = plsc.load_gather(idx_v, [rid + 2])
            xv = plsc.load_gather(idx_v, [rid + 3])
            lin = zv * _NHW + (bv * _H + yv) * _W + xv
            lin = jnp.where(valid, lin, 2 * _NHW + wid * _CH + s * 16 + lane)
            lin_v[j, pl.ds(s * 16, 16)] = lin
            mlin_v[j, pl.ds(s * 16, 16)] = core * _GROWS + lin
            rowid_v[j, pl.ds(s * 16, 16)] = jnp.where(valid, v0 + lid, 0)
        return 0

    lax.fori_loop(0, _NCH, chunk, 0)

    for d in zd:
        d.wait()
    plsc.subcore_barrier()

    md = [pltpu.async_copy(ones_v, mask_hbm.at[mlin_v.at[j]], msem)
          for j in range(_NCH)]

    gd = [None] * _NCH
    sd = [None] * _NCH

    def fire_gather(j):
        return pltpu.async_copy(features_hbm.at[rowid_v.at[j]],
                                fbuf.at[j % _RING], gsems[j % _RING])

    for j in range(_RING):
        gd[j] = fire_gather(j)
    for j in range(_NCH):
        gd[j].wait()
        sd[j] = pltpu.async_copy(fbuf.at[j % _RING], g_hbm.at[lin_v.at[j]],
                                 ssems[j % _RING])
        if j + _RING < _NCH:
            sd[j].wait()
            gd[j + _RING] = fire_gather(j + _RING)
    for j in range(_NCH - _RING, _NCH):
        sd[j].wait()
    for d in md:
        d.wait()


@functools.partial(
    pl.kernel,
    out_type=(
        jax.ShapeDtypeStruct((_GROWS, _C), jnp.float32),
        jax.ShapeDtypeStruct((_MASKLEN,), jnp.float32),
    ),
    mesh=plsc.VectorSubcoreMesh(core_axis_name="c", subcore_axis_name="s"),
    compiler_params=pltpu.CompilerParams(needs_layout_passes=False),
    scratch_types=[
        pltpu.VMEM((2048 * 4,), jnp.int32),
        pltpu.VMEM((_RING, _CH, _C), jnp.float32),
        pltpu.VMEM((_NCH, _CH), jnp.int32),
        pltpu.VMEM((_NCH, _CH), jnp.int32),
        pltpu.VMEM((_NCH, _CH), jnp.int32),
        pltpu.VMEM((_CH,), jnp.float32),
        pltpu.VMEM((_ZB,), jnp.float32),
        pltpu.SemaphoreType.DMA,
        pltpu.SemaphoreType.DMA,
        [pltpu.SemaphoreType.DMA] * _RING,
        [pltpu.SemaphoreType.DMA] * _RING,
    ],
)
def _sc_scatter(features_hbm, indices_hbm, g_hbm, mask_hbm, *scratch):
    _sc_body(features_hbm, indices_hbm, g_hbm, mask_hbm, *scratch)


def _tc_body(se_ref, so_ref, m0a_ref, m0b_ref, m1a_ref, m1b_ref,
             e_ref, o_ref, o_out_ref):
    m0 = m0a_ref[...] + m0b_ref[...]
    m1 = m1a_ref[...] + m1b_ref[...]
    em = e_ref[...] * m0[:, None]
    om = o_ref[...] * m1[:, None]
    o_out_ref[...] = (
        jax.lax.dot(em, se_ref[0], preferred_element_type=jnp.float32)
        + jax.lax.dot(om, so_ref[0], preferred_element_type=jnp.float32)
    )


def kernel(features, indices):
    g, mask = _sc_scatter(features, indices.reshape(-1))
    row = jnp.arange(_C)[None, :, None]
    col = jnp.arange(2 * _C)[None, None, :]
    par = jnp.arange(2)[:, None, None]
    sel = (col == 2 * row + par).astype(jnp.float32)
    nblk = _NHW // _CB
    zoff = _NHW // _CB
    coff = _GROWS // _CB
    out_p = pl.pallas_call(
        _tc_body,
        grid=(nblk,),
        in_specs=[
            pl.BlockSpec((1, _C, 2 * _C), lambda j: (0, 0, 0)),
            pl.BlockSpec((1, _C, 2 * _C), lambda j: (1, 0, 0)),
            pl.BlockSpec((_CB,), lambda j: (j,)),
            pl.BlockSpec((_CB,), lambda j: (coff + j,)),
            pl.BlockSpec((_CB,), lambda j: (zoff + j,)),
            pl.BlockSpec((_CB,), lambda j: (coff + zoff + j,)),
            pl.BlockSpec((_CB, _C), lambda j: (j, 0)),
            pl.BlockSpec((_CB, _C), lambda j: (zoff + j, 0)),
        ],
        out_specs=pl.BlockSpec((_CB, 2 * _C), lambda j: (j, 0)),
        out_shape=jax.ShapeDtypeStruct((_NHW, 2 * _C), jnp.float32),
    )(sel, sel, mask, mask, mask, mask, g, g)
    return out_p.reshape(_N, _H, _W, _C * _D).transpose(0, 3, 1, 2)


# speedup vs baseline: 3.3659x; 1.3672x over previous
import functools

import jax
import jax.numpy as jnp
from jax import lax
from jax.experimental import pallas as pl
from jax.experimental.pallas import tpu as pltpu
from jax.experimental.pallas import tpu_sc as plsc

_N, _C, _D, _H, _W = 4, 128, 2, 200, 176
_NNZ = 60000
_NHW = _N * _H * _W
_NWORK = 32
_PER_W = 1872
_LAST_W = _NNZ - 31 * _PER_W
_NCH = 16
_CH = 128
_RING = 5
_LEAD = 3
_SPARE = _NWORK * _CH
_GROWS = 2 * _NHW + _SPARE
_MASKLEN = 2 * _GROWS
_MZ_PER_TILE = _GROWS // 16
_ZB = 2976
_CB = 256


def _sc_body(features_hbm, indices_hbm, g_hbm, mask_hbm,
             idx_v, fbuf, lin_v, mlin_v, ones_v, zbuf,
             zsem, msem, gsems, ssems):
    core = lax.axis_index("c")
    sub = lax.axis_index("s")
    wid = sub * 2 + core
    v0 = wid * _PER_W
    nw = jnp.where(wid == _NWORK - 1, _LAST_W, _PER_W)

    zeros16 = jnp.zeros((16,), jnp.float32)

    def fill(i, _):
        zbuf[pl.ds(i * 16, 16)] = zeros16
        return 0
    lax.fori_loop(0, _ZB // 16, fill, 0)

    def fill1(i, _):
        ones_v[pl.ds(i * 16, 16)] = zeros16 + 1.0
        return 0
    lax.fori_loop(0, _CH // 16, fill1, 0)

    zd = []
    mbase = core * _GROWS + sub * _MZ_PER_TILE
    for k in range(_MZ_PER_TILE // _ZB):
        zd.append(pltpu.async_copy(
            zbuf, mask_hbm.at[pl.ds(mbase + k * _ZB, _ZB)], zsem))

    pltpu.sync_copy(indices_hbm.at[pl.ds(v0 * 4, _LAST_W * 4)],
                    idx_v.at[pl.ds(0, _LAST_W * 4)])

    lane = lax.iota(jnp.int32, 16)

    def chunk(j, _):
        src0 = jnp.minimum(v0 + j * _CH, _NNZ - _CH)
        for s in range(_CH // 16):
            lid = src0 - v0 + s * 16 + lane
            valid = lid < nw
            rid = jnp.where(valid, lid, 0) * 4
            bv = plsc.load_gather(idx_v, [rid])
            zv = plsc.load_gather(idx_v, [rid + 1])
            yv = plsc.load_gather(idx_v, [rid + 2])
            xv = plsc.load_gather(idx_v, [rid + 3])
            lin = zv * _NHW + (bv * _H + yv) * _W + xv
            lin = jnp.where(valid, lin, 2 * _NHW + wid * _CH + s * 16 + lane)
            lin_v[j, pl.ds(s * 16, 16)] = lin
            mlin_v[j, pl.ds(s * 16, 16)] = core * _GROWS + lin
        return 0

    lax.fori_loop(0, _NCH, chunk, 0)

    for d in zd:
        d.wait()
    plsc.subcore_barrier()

    md = [pltpu.async_copy(ones_v, mask_hbm.at[mlin_v.at[j]], msem)
          for j in range(_NCH)]

    gd = [None] * _NCH
    sd = [None] * _NCH

    def fire_gather(j):
        start = jnp.minimum(v0 + j * _CH, _NNZ - _CH)
        return pltpu.async_copy(
            features_hbm.at[pl.ds(start, _CH)], fbuf.at[j % _RING],
            gsems[j % _RING])

    def fire_scatter(j):
        return pltpu.async_copy(fbuf.at[j % _RING], g_hbm.at[lin_v.at[j]],
                                ssems[j % _RING])

    for j in range(_NCH + _LEAD):
        if j < _NCH:
            if j >= _RING:
                sd[j - _RING].wait()
            gd[j] = fire_gather(j)
        if j >= _LEAD:
            gd[j - _LEAD].wait()
            sd[j - _LEAD] = fire_scatter(j - _LEAD)
    for j in range(_NCH - _RING, _NCH):
        sd[j].wait()
    for d in md:
        d.wait()


@functools.partial(
    pl.kernel,
    out_type=(
        jax.ShapeDtypeStruct((_GROWS, _C), jnp.float32),
        jax.ShapeDtypeStruct((_MASKLEN,), jnp.float32),
    ),
    mesh=plsc.VectorSubcoreMesh(core_axis_name="c", subcore_axis_name="s"),
    compiler_params=pltpu.CompilerParams(needs_layout_passes=False),
    scratch_types=[
        pltpu.VMEM((2048 * 4,), jnp.int32),
        pltpu.VMEM((_RING, _CH, _C), jnp.float32),
        pltpu.VMEM((_NCH, _CH), jnp.int32),
        pltpu.VMEM((_NCH, _CH), jnp.int32),
        pltpu.VMEM((_CH,), jnp.float32),
        pltpu.VMEM((_ZB,), jnp.float32),
        pltpu.SemaphoreType.DMA,
        pltpu.SemaphoreType.DMA,
        [pltpu.SemaphoreType.DMA] * _RING,
        [pltpu.SemaphoreType.DMA] * _RING,
    ],
)
def _sc_scatter(features_hbm, indices_hbm, g_hbm, mask_hbm, *scratch):
    _sc_body(features_hbm, indices_hbm, g_hbm, mask_hbm, *scratch)


def _tc_body(se_ref, so_ref, m0a_ref, m0b_ref, m1a_ref, m1b_ref,
             e_ref, o_ref, o_out_ref):
    m0 = m0a_ref[...] + m0b_ref[...]
    m1 = m1a_ref[...] + m1b_ref[...]
    em = e_ref[...] * m0[:, None]
    om = o_ref[...] * m1[:, None]
    o_out_ref[...] = (
        jax.lax.dot(em, se_ref[0], preferred_element_type=jnp.float32)
        + jax.lax.dot(om, so_ref[0], preferred_element_type=jnp.float32)
    )


def kernel(features, indices):
    g, mask = _sc_scatter(features, indices.reshape(-1))
    row = jnp.arange(_C)[None, :, None]
    col = jnp.arange(2 * _C)[None, None, :]
    par = jnp.arange(2)[:, None, None]
    sel = (col == 2 * row + par).astype(jnp.float32)
    nblk = _NHW // _CB
    zoff = _NHW // _CB
    coff = _GROWS // _CB
    out_p = pl.pallas_call(
        _tc_body,
        grid=(nblk,),
        in_specs=[
            pl.BlockSpec((1, _C, 2 * _C), lambda j: (0, 0, 0)),
            pl.BlockSpec((1, _C, 2 * _C), lambda j: (1, 0, 0)),
            pl.BlockSpec((_CB,), lambda j: (j,)),
            pl.BlockSpec((_CB,), lambda j: (coff + j,)),
            pl.BlockSpec((_CB,), lambda j: (zoff + j,)),
            pl.BlockSpec((_CB,), lambda j: (coff + zoff + j,)),
            pl.BlockSpec((_CB, _C), lambda j: (j, 0)),
            pl.BlockSpec((_CB, _C), lambda j: (zoff + j, 0)),
        ],
        out_specs=pl.BlockSpec((_CB, 2 * _C), lambda j: (j, 0)),
        out_shape=jax.ShapeDtypeStruct((_NHW, 2 * _C), jnp.float32),
    )(sel, sel, mask, mask, mask, mask, g, g)
    return out_p.reshape(_N, _H, _W, _C * _D).transpose(0, 3, 1, 2)


# speedup vs baseline: 4.6236x; 1.3737x over previous
import functools

import jax
import jax.numpy as jnp
from jax import lax
from jax.experimental import pallas as pl
from jax.experimental.pallas import tpu as pltpu
from jax.experimental.pallas import tpu_sc as plsc

_N, _C, _D, _H, _W = 4, 128, 2, 200, 176
_NNZ = 60000
_NHW = _N * _H * _W
_NWORK = 32
_PER_W = 1872
_LAST_W = _NNZ - 31 * _PER_W
_NCH = 16
_CH = 128
_RING = 5
_LEAD = 3
_SPARE = _NWORK * _CH
_GROWS = 2 * _NHW + _SPARE
_MASKLEN = 2 * _GROWS
_MZ_PER_TILE = _GROWS // 16
_ZB = 2976
_CB = 512


def _sc_body(features_hbm, indices_hbm, g_hbm, mask_hbm,
             idx_v, fbuf, lin_v, mlin_v, ones_v, zbuf,
             zsem, msem, gsems, ssems):
    core = lax.axis_index("c")
    sub = lax.axis_index("s")
    wid = sub * 2 + core
    v0 = wid * _PER_W
    nw = jnp.where(wid == _NWORK - 1, _LAST_W, _PER_W)

    zeros16 = jnp.zeros((16,), jnp.float32)

    def fill(i, _):
        zbuf[pl.ds(i * 16, 16)] = zeros16
        return 0
    lax.fori_loop(0, _ZB // 16, fill, 0)

    def fill1(i, _):
        ones_v[pl.ds(i * 16, 16)] = zeros16 + 1.0
        return 0
    lax.fori_loop(0, _CH // 16, fill1, 0)

    zd = []
    mbase = core * _GROWS + sub * _MZ_PER_TILE
    for k in range(_MZ_PER_TILE // _ZB):
        zd.append(pltpu.async_copy(
            zbuf, mask_hbm.at[pl.ds(mbase + k * _ZB, _ZB)], zsem))

    pltpu.sync_copy(indices_hbm.at[pl.ds(v0 * 4, _LAST_W * 4)],
                    idx_v.at[pl.ds(0, _LAST_W * 4)])

    lane = lax.iota(jnp.int32, 16)

    def chunk(j, _):
        src0 = jnp.minimum(v0 + j * _CH, _NNZ - _CH)
        for s in range(_CH // 16):
            lid = src0 - v0 + s * 16 + lane
            valid = lid < nw
            rid = jnp.where(valid, lid, 0) * 4
            bv = plsc.load_gather(idx_v, [rid])
            zv = plsc.load_gather(idx_v, [rid + 1])
            yv = plsc.load_gather(idx_v, [rid + 2])
            xv = plsc.load_gather(idx_v, [rid + 3])
            lin = zv * _NHW + (bv * _H + yv) * _W + xv
            lin = jnp.where(valid, lin, 2 * _NHW + wid * _CH + s * 16 + lane)
            lin_v[j, pl.ds(s * 16, 16)] = lin
            mlin_v[j, pl.ds(s * 16, 16)] = core * _GROWS + lin
        return 0

    lax.fori_loop(0, _NCH, chunk, 0)

    for d in zd:
        d.wait()
    plsc.subcore_barrier()

    md = [pltpu.async_copy(ones_v, mask_hbm.at[mlin_v.at[j]], msem)
          for j in range(_NCH)]

    gd = [None] * _NCH
    sd = [None] * _NCH

    def fire_gather(j):
        start = jnp.minimum(v0 + j * _CH, _NNZ - _CH)
        return pltpu.async_copy(
            features_hbm.at[pl.ds(start, _CH)], fbuf.at[j % _RING],
            gsems[j % _RING])

    def fire_scatter(j):
        return pltpu.async_copy(fbuf.at[j % _RING], g_hbm.at[lin_v.at[j]],
                                ssems[j % _RING])

    for j in range(_NCH + _LEAD):
        if j < _NCH:
            if j >= _RING:
                sd[j - _RING].wait()
            gd[j] = fire_gather(j)
        if j >= _LEAD:
            gd[j - _LEAD].wait()
            sd[j - _LEAD] = fire_scatter(j - _LEAD)
    for j in range(_NCH - _RING, _NCH):
        sd[j].wait()
    for d in md:
        d.wait()


@functools.partial(
    pl.kernel,
    out_type=(
        jax.ShapeDtypeStruct((_GROWS, _C), jnp.float32),
        jax.ShapeDtypeStruct((_MASKLEN,), jnp.float32),
    ),
    mesh=plsc.VectorSubcoreMesh(core_axis_name="c", subcore_axis_name="s"),
    compiler_params=pltpu.CompilerParams(needs_layout_passes=False),
    scratch_types=[
        pltpu.VMEM((2048 * 4,), jnp.int32),
        pltpu.VMEM((_RING, _CH, _C), jnp.float32),
        pltpu.VMEM((_NCH, _CH), jnp.int32),
        pltpu.VMEM((_NCH, _CH), jnp.int32),
        pltpu.VMEM((_CH,), jnp.float32),
        pltpu.VMEM((_ZB,), jnp.float32),
        pltpu.SemaphoreType.DMA,
        pltpu.SemaphoreType.DMA,
        [pltpu.SemaphoreType.DMA] * _RING,
        [pltpu.SemaphoreType.DMA] * _RING,
    ],
)
def _sc_scatter(features_hbm, indices_hbm, g_hbm, mask_hbm, *scratch):
    _sc_body(features_hbm, indices_hbm, g_hbm, mask_hbm, *scratch)


def _tc_body(se_ref, so_ref, m0a_ref, m0b_ref, m1a_ref, m1b_ref,
             e_ref, o_ref, o_out_ref):
    m0 = m0a_ref[...] + m0b_ref[...]
    m1 = m1a_ref[...] + m1b_ref[...]
    em = e_ref[...] * m0[:, None]
    om = o_ref[...] * m1[:, None]
    o_out_ref[...] = (
        jax.lax.dot(em, se_ref[0], preferred_element_type=jnp.float32)
        + jax.lax.dot(om, so_ref[0], preferred_element_type=jnp.float32)
    )


def kernel(features, indices):
    g, mask = _sc_scatter(features, indices.reshape(-1))
    row = jnp.arange(_C)[None, :, None]
    col = jnp.arange(2 * _C)[None, None, :]
    par = jnp.arange(2)[:, None, None]
    sel = (col == 2 * row + par).astype(jnp.float32)
    nblk = _NHW // _CB
    zoff = _NHW // _CB
    coff = _GROWS // _CB
    out_p = pl.pallas_call(
        _tc_body,
        grid=(nblk,),
        in_specs=[
            pl.BlockSpec((1, _C, 2 * _C), lambda j: (0, 0, 0)),
            pl.BlockSpec((1, _C, 2 * _C), lambda j: (1, 0, 0)),
            pl.BlockSpec((_CB,), lambda j: (j,)),
            pl.BlockSpec((_CB,), lambda j: (coff + j,)),
            pl.BlockSpec((_CB,), lambda j: (zoff + j,)),
            pl.BlockSpec((_CB,), lambda j: (coff + zoff + j,)),
            pl.BlockSpec((_CB, _C), lambda j: (j, 0)),
            pl.BlockSpec((_CB, _C), lambda j: (zoff + j, 0)),
        ],
        out_specs=pl.BlockSpec((_CB, 2 * _C), lambda j: (j, 0)),
        out_shape=jax.ShapeDtypeStruct((_NHW, 2 * _C), jnp.float32),
    )(sel, sel, mask, mask, mask, mask, g, g)
    return out_p.reshape(_N, _H, _W, _C * _D).transpose(0, 3, 1, 2)


# speedup vs baseline: 5.6563x; 1.2233x over previous
import functools

import jax
import jax.numpy as jnp
from jax import lax
from jax.experimental import pallas as pl
from jax.experimental.pallas import tpu as pltpu
from jax.experimental.pallas import tpu_sc as plsc

_N, _C, _D, _H, _W = 4, 128, 2, 200, 176
_NNZ = 60000
_NHW = _N * _H * _W
_NWORK = 32
_PER_W = 1872
_LAST_W = _NNZ - 31 * _PER_W
_NCH = 16
_CH = 128
_RING = 5
_LEAD = 3
_SPARE = _NWORK * _CH
_GROWS = 2 * _NHW + _SPARE
_MASKLEN = 2 * _GROWS
_MZ_PER_TILE = _GROWS // 16
_ZB = 2976
_CB = 1024


def _sc_body(features_hbm, indices_hbm, g_hbm, mask_hbm,
             idx_v, fbuf, lin_v, mlin_v, ones_v, zbuf,
             zsem, msem, gsems, ssems):
    core = lax.axis_index("c")
    sub = lax.axis_index("s")
    wid = sub * 2 + core
    v0 = wid * _PER_W
    nw = jnp.where(wid == _NWORK - 1, _LAST_W, _PER_W)

    zeros16 = jnp.zeros((16,), jnp.float32)

    def fill(i, _):
        zbuf[pl.ds(i * 16, 16)] = zeros16
        return 0
    lax.fori_loop(0, _ZB // 16, fill, 0)

    def fill1(i, _):
        ones_v[pl.ds(i * 16, 16)] = zeros16 + 1.0
        return 0
    lax.fori_loop(0, _CH // 16, fill1, 0)

    zd = []
    mbase = core * _GROWS + sub * _MZ_PER_TILE
    for k in range(_MZ_PER_TILE // _ZB):
        zd.append(pltpu.async_copy(
            zbuf, mask_hbm.at[pl.ds(mbase + k * _ZB, _ZB)], zsem))

    pltpu.sync_copy(indices_hbm.at[pl.ds(v0 * 4, _LAST_W * 4)],
                    idx_v.at[pl.ds(0, _LAST_W * 4)])

    lane = lax.iota(jnp.int32, 16)

    def chunk(j, _):
        src0 = jnp.minimum(v0 + j * _CH, _NNZ - _CH)
        for s in range(_CH // 16):
            lid = src0 - v0 + s * 16 + lane
            valid = lid < nw
            rid = jnp.where(valid, lid, 0) * 4
            bv = plsc.load_gather(idx_v, [rid])
            zv = plsc.load_gather(idx_v, [rid + 1])
            yv = plsc.load_gather(idx_v, [rid + 2])
            xv = plsc.load_gather(idx_v, [rid + 3])
            lin = zv * _NHW + (bv * _H + yv) * _W + xv
            lin = jnp.where(valid, lin, 2 * _NHW + wid * _CH + s * 16 + lane)
            lin_v[j, pl.ds(s * 16, 16)] = lin
            mlin_v[j, pl.ds(s * 16, 16)] = core * _GROWS + lin
        return 0

    lax.fori_loop(0, _NCH, chunk, 0)

    for d in zd:
        d.wait()
    plsc.subcore_barrier()

    md = [pltpu.async_copy(ones_v, mask_hbm.at[mlin_v.at[j]], msem)
          for j in range(_NCH)]

    gd = [None] * _NCH
    sd = [None] * _NCH

    def fire_gather(j):
        start = jnp.minimum(v0 + j * _CH, _NNZ - _CH)
        return pltpu.async_copy(
            features_hbm.at[pl.ds(start, _CH)], fbuf.at[j % _RING],
            gsems[j % _RING])

    def fire_scatter(j):
        return pltpu.async_copy(fbuf.at[j % _RING], g_hbm.at[lin_v.at[j]],
                                ssems[j % _RING])

    for j in range(_NCH + _LEAD):
        if j < _NCH:
            if j >= _RING:
                sd[j - _RING].wait()
            gd[j] = fire_gather(j)
        if j >= _LEAD:
            gd[j - _LEAD].wait()
            sd[j - _LEAD] = fire_scatter(j - _LEAD)
    for j in range(_NCH - _RING, _NCH):
        sd[j].wait()
    for d in md:
        d.wait()


@functools.partial(
    pl.kernel,
    out_type=(
        jax.ShapeDtypeStruct((_GROWS, _C), jnp.float32),
        jax.ShapeDtypeStruct((_MASKLEN,), jnp.float32),
    ),
    mesh=plsc.VectorSubcoreMesh(core_axis_name="c", subcore_axis_name="s"),
    compiler_params=pltpu.CompilerParams(needs_layout_passes=False),
    scratch_types=[
        pltpu.VMEM((2048 * 4,), jnp.int32),
        pltpu.VMEM((_RING, _CH, _C), jnp.float32),
        pltpu.VMEM((_NCH, _CH), jnp.int32),
        pltpu.VMEM((_NCH, _CH), jnp.int32),
        pltpu.VMEM((_CH,), jnp.float32),
        pltpu.VMEM((_ZB,), jnp.float32),
        pltpu.SemaphoreType.DMA,
        pltpu.SemaphoreType.DMA,
        [pltpu.SemaphoreType.DMA] * _RING,
        [pltpu.SemaphoreType.DMA] * _RING,
    ],
)
def _sc_scatter(features_hbm, indices_hbm, g_hbm, mask_hbm, *scratch):
    _sc_body(features_hbm, indices_hbm, g_hbm, mask_hbm, *scratch)


def _tc_body(se_ref, so_ref, m0a_ref, m0b_ref, m1a_ref, m1b_ref,
             e_ref, o_ref, o_out_ref):
    m0 = m0a_ref[...] + m0b_ref[...]
    m1 = m1a_ref[...] + m1b_ref[...]
    em = e_ref[...] * m0[:, None]
    om = o_ref[...] * m1[:, None]
    o_out_ref[...] = (
        jax.lax.dot(em, se_ref[0], preferred_element_type=jnp.float32)
        + jax.lax.dot(om, so_ref[0], preferred_element_type=jnp.float32)
    )


def kernel(features, indices):
    g, mask = _sc_scatter(features, indices.reshape(-1))
    row = jnp.arange(_C)[None, :, None]
    col = jnp.arange(2 * _C)[None, None, :]
    par = jnp.arange(2)[:, None, None]
    sel = (col == 2 * row + par).astype(jnp.float32)
    nblk = _NHW // _CB
    zoff = _NHW // _CB
    coff = _GROWS // _CB
    out_p = pl.pallas_call(
        _tc_body,
        grid=(nblk,),
        in_specs=[
            pl.BlockSpec((1, _C, 2 * _C), lambda j: (0, 0, 0)),
            pl.BlockSpec((1, _C, 2 * _C), lambda j: (1, 0, 0)),
            pl.BlockSpec((_CB,), lambda j: (j,)),
            pl.BlockSpec((_CB,), lambda j: (coff + j,)),
            pl.BlockSpec((_CB,), lambda j: (zoff + j,)),
            pl.BlockSpec((_CB,), lambda j: (coff + zoff + j,)),
            pl.BlockSpec((_CB, _C), lambda j: (j, 0)),
            pl.BlockSpec((_CB, _C), lambda j: (zoff + j, 0)),
        ],
        out_specs=pl.BlockSpec((_CB, 2 * _C), lambda j: (j, 0)),
        out_shape=jax.ShapeDtypeStruct((_NHW, 2 * _C), jnp.float32),
    )(sel, sel, mask, mask, mask, mask, g, g)
    return out_p.reshape(_N, _H, _W, _C * _D).transpose(0, 3, 1, 2)


# speedup vs baseline: 6.5735x; 1.1622x over previous
import functools

import jax
import jax.numpy as jnp
from jax import lax
from jax.experimental import pallas as pl
from jax.experimental.pallas import tpu as pltpu
from jax.experimental.pallas import tpu_sc as plsc

_N, _C, _D, _H, _W = 4, 128, 2, 200, 176
_NNZ = 60000
_NHW = _N * _H * _W
_NWORK = 32
_PER_W = 1872
_LAST_W = _NNZ - 31 * _PER_W
_NCH = 16
_CH = 128
_RING = 5
_LEAD = 3
_SPARE = _NWORK * _CH
_GROWS = 2 * _NHW + _SPARE
_MASKLEN = 2 * _GROWS
_MZ_PER_TILE = _GROWS // 16
_ZB = 2976
_CB = 2048


def _sc_body(features_hbm, indices_hbm, g_hbm, mask_hbm,
             idx_v, fbuf, lin_v, mlin_v, ones_v, zbuf,
             zsem, msem, gsems, ssems):
    core = lax.axis_index("c")
    sub = lax.axis_index("s")
    wid = sub * 2 + core
    v0 = wid * _PER_W
    nw = jnp.where(wid == _NWORK - 1, _LAST_W, _PER_W)

    zeros16 = jnp.zeros((16,), jnp.float32)

    def fill(i, _):
        zbuf[pl.ds(i * 16, 16)] = zeros16
        return 0
    lax.fori_loop(0, _ZB // 16, fill, 0)

    def fill1(i, _):
        ones_v[pl.ds(i * 16, 16)] = zeros16 + 1.0
        return 0
    lax.fori_loop(0, _CH // 16, fill1, 0)

    zd = []
    mbase = core * _GROWS + sub * _MZ_PER_TILE
    for k in range(_MZ_PER_TILE // _ZB):
        zd.append(pltpu.async_copy(
            zbuf, mask_hbm.at[pl.ds(mbase + k * _ZB, _ZB)], zsem))

    pltpu.sync_copy(indices_hbm.at[pl.ds(v0 * 4, _LAST_W * 4)],
                    idx_v.at[pl.ds(0, _LAST_W * 4)])

    lane = lax.iota(jnp.int32, 16)

    def chunk(j, _):
        src0 = jnp.minimum(v0 + j * _CH, _NNZ - _CH)
        for s in range(_CH // 16):
            lid = src0 - v0 + s * 16 + lane
            valid = lid < nw
            rid = jnp.where(valid, lid, 0) * 4
            bv = plsc.load_gather(idx_v, [rid])
            zv = plsc.load_gather(idx_v, [rid + 1])
            yv = plsc.load_gather(idx_v, [rid + 2])
            xv = plsc.load_gather(idx_v, [rid + 3])
            lin = zv * _NHW + (bv * _H + yv) * _W + xv
            lin = jnp.where(valid, lin, 2 * _NHW + wid * _CH + s * 16 + lane)
            lin_v[j, pl.ds(s * 16, 16)] = lin
            mlin_v[j, pl.ds(s * 16, 16)] = core * _GROWS + lin
        return 0

    lax.fori_loop(0, _NCH, chunk, 0)

    for d in zd:
        d.wait()
    plsc.subcore_barrier()

    md = [pltpu.async_copy(ones_v, mask_hbm.at[mlin_v.at[j]], msem)
          for j in range(_NCH)]

    gd = [None] * _NCH
    sd = [None] * _NCH

    def fire_gather(j):
        start = jnp.minimum(v0 + j * _CH, _NNZ - _CH)
        return pltpu.async_copy(
            features_hbm.at[pl.ds(start, _CH)], fbuf.at[j % _RING],
            gsems[j % _RING])

    def fire_scatter(j):
        return pltpu.async_copy(fbuf.at[j % _RING], g_hbm.at[lin_v.at[j]],
                                ssems[j % _RING])

    for j in range(_NCH + _LEAD):
        if j < _NCH:
            if j >= _RING:
                sd[j - _RING].wait()
            gd[j] = fire_gather(j)
        if j >= _LEAD:
            gd[j - _LEAD].wait()
            sd[j - _LEAD] = fire_scatter(j - _LEAD)
    for j in range(_NCH - _RING, _NCH):
        sd[j].wait()
    for d in md:
        d.wait()


@functools.partial(
    pl.kernel,
    out_type=(
        jax.ShapeDtypeStruct((_GROWS, _C), jnp.float32),
        jax.ShapeDtypeStruct((_MASKLEN,), jnp.float32),
    ),
    mesh=plsc.VectorSubcoreMesh(core_axis_name="c", subcore_axis_name="s"),
    compiler_params=pltpu.CompilerParams(needs_layout_passes=False),
    scratch_types=[
        pltpu.VMEM((2048 * 4,), jnp.int32),
        pltpu.VMEM((_RING, _CH, _C), jnp.float32),
        pltpu.VMEM((_NCH, _CH), jnp.int32),
        pltpu.VMEM((_NCH, _CH), jnp.int32),
        pltpu.VMEM((_CH,), jnp.float32),
        pltpu.VMEM((_ZB,), jnp.float32),
        pltpu.SemaphoreType.DMA,
        pltpu.SemaphoreType.DMA,
        [pltpu.SemaphoreType.DMA] * _RING,
        [pltpu.SemaphoreType.DMA] * _RING,
    ],
)
def _sc_scatter(features_hbm, indices_hbm, g_hbm, mask_hbm, *scratch):
    _sc_body(features_hbm, indices_hbm, g_hbm, mask_hbm, *scratch)


def _tc_body(se_ref, so_ref, m0a_ref, m0b_ref, m1a_ref, m1b_ref,
             e_ref, o_ref, o_out_ref):
    m0 = m0a_ref[...] + m0b_ref[...]
    m1 = m1a_ref[...] + m1b_ref[...]
    em = e_ref[...] * m0[:, None]
    om = o_ref[...] * m1[:, None]
    o_out_ref[...] = (
        jax.lax.dot(em, se_ref[0], preferred_element_type=jnp.float32)
        + jax.lax.dot(om, so_ref[0], preferred_element_type=jnp.float32)
    )


def kernel(features, indices):
    g, mask = _sc_scatter(features, indices.reshape(-1))
    row = jnp.arange(_C)[None, :, None]
    col = jnp.arange(2 * _C)[None, None, :]
    par = jnp.arange(2)[:, None, None]
    sel = (col == 2 * row + par).astype(jnp.float32)
    nblk = _NHW // _CB
    zoff = _NHW // _CB
    coff = _GROWS // _CB
    out_p = pl.pallas_call(
        _tc_body,
        grid=(nblk,),
        in_specs=[
            pl.BlockSpec((1, _C, 2 * _C), lambda j: (0, 0, 0)),
            pl.BlockSpec((1, _C, 2 * _C), lambda j: (1, 0, 0)),
            pl.BlockSpec((_CB,), lambda j: (j,)),
            pl.BlockSpec((_CB,), lambda j: (coff + j,)),
            pl.BlockSpec((_CB,), lambda j: (zoff + j,)),
            pl.BlockSpec((_CB,), lambda j: (coff + zoff + j,)),
            pl.BlockSpec((_CB, _C), lambda j: (j, 0)),
            pl.BlockSpec((_CB, _C), lambda j: (zoff + j, 0)),
        ],
        out_specs=pl.BlockSpec((_CB, 2 * _C), lambda j: (j, 0)),
        out_shape=jax.ShapeDtypeStruct((_NHW, 2 * _C), jnp.float32),
    )(sel, sel, mask, mask, mask, mask, g, g)
    return out_p.reshape(_N, _H, _W, _C * _D).transpose(0, 3, 1, 2)


# speedup vs baseline: 6.7005x; 1.0193x over previous
import functools

import jax
import jax.numpy as jnp
from jax import lax
from jax.experimental import pallas as pl
from jax.experimental.pallas import tpu as pltpu
from jax.experimental.pallas import tpu_sc as plsc

_N, _C, _D, _H, _W = 4, 128, 2, 200, 176
_NNZ = 60000
_NHW = _N * _H * _W
_NWORK = 32
_PER_W = 1872
_LAST_W = _NNZ - 31 * _PER_W
_NCH = 16
_CH = 128
_RING = 5
_LEAD = 3
_SPARE = 5120
_GROWS = 2 * _NHW + _SPARE
_MASKLEN = 2 * _GROWS
_MZ_PER_TILE = _GROWS // 16
_ZB = 2240
_CB = 2560


def _sc_body(features_hbm, indices_hbm, g_hbm, mask_hbm,
             idx_v, fbuf, lin_v, mlin_v, ones_v, zbuf,
             zsem, msem, gsems, ssems):
    core = lax.axis_index("c")
    sub = lax.axis_index("s")
    wid = sub * 2 + core
    v0 = wid * _PER_W
    nw = jnp.where(wid == _NWORK - 1, _LAST_W, _PER_W)

    zeros16 = jnp.zeros((16,), jnp.float32)

    def fill(i, _):
        zbuf[pl.ds(i * 16, 16)] = zeros16
        return 0
    lax.fori_loop(0, _ZB // 16, fill, 0)

    def fill1(i, _):
        ones_v[pl.ds(i * 16, 16)] = zeros16 + 1.0
        return 0
    lax.fori_loop(0, _CH // 16, fill1, 0)

    zd = []
    mbase = core * _GROWS + sub * _MZ_PER_TILE
    for k in range(_MZ_PER_TILE // _ZB):
        zd.append(pltpu.async_copy(
            zbuf, mask_hbm.at[pl.ds(mbase + k * _ZB, _ZB)], zsem))

    pltpu.sync_copy(indices_hbm.at[pl.ds(v0 * 4, _LAST_W * 4)],
                    idx_v.at[pl.ds(0, _LAST_W * 4)])

    lane = lax.iota(jnp.int32, 16)

    def chunk(j, _):
        src0 = jnp.minimum(v0 + j * _CH, _NNZ - _CH)
        for s in range(_CH // 16):
            lid = src0 - v0 + s * 16 + lane
            valid = lid < nw
            rid = jnp.where(valid, lid, 0) * 4
            bv = plsc.load_gather(idx_v, [rid])
            zv = plsc.load_gather(idx_v, [rid + 1])
            yv = plsc.load_gather(idx_v, [rid + 2])
            xv = plsc.load_gather(idx_v, [rid + 3])
            lin = zv * _NHW + (bv * _H + yv) * _W + xv
            lin = jnp.where(valid, lin, 2 * _NHW + wid * _CH + s * 16 + lane)
            lin_v[j, pl.ds(s * 16, 16)] = lin
            mlin_v[j, pl.ds(s * 16, 16)] = core * _GROWS + lin
        return 0

    lax.fori_loop(0, _NCH, chunk, 0)

    for d in zd:
        d.wait()
    plsc.subcore_barrier()

    md = [pltpu.async_copy(ones_v, mask_hbm.at[mlin_v.at[j]], msem)
          for j in range(_NCH)]

    gd = [None] * _NCH
    sd = [None] * _NCH

    def fire_gather(j):
        start = jnp.minimum(v0 + j * _CH, _NNZ - _CH)
        return pltpu.async_copy(
            features_hbm.at[pl.ds(start, _CH)], fbuf.at[j % _RING],
            gsems[j % _RING])

    def fire_scatter(j):
        return pltpu.async_copy(fbuf.at[j % _RING], g_hbm.at[lin_v.at[j]],
                                ssems[j % _RING])

    for j in range(_NCH + _LEAD):
        if j < _NCH:
            if j >= _RING:
                sd[j - _RING].wait()
            gd[j] = fire_gather(j)
        if j >= _LEAD:
            gd[j - _LEAD].wait()
            sd[j - _LEAD] = fire_scatter(j - _LEAD)
    for j in range(_NCH - _RING, _NCH):
        sd[j].wait()
    for d in md:
        d.wait()


@functools.partial(
    pl.kernel,
    out_type=(
        jax.ShapeDtypeStruct((_GROWS, _C), jnp.float32),
        jax.ShapeDtypeStruct((_MASKLEN,), jnp.float32),
    ),
    mesh=plsc.VectorSubcoreMesh(core_axis_name="c", subcore_axis_name="s"),
    compiler_params=pltpu.CompilerParams(needs_layout_passes=False),
    scratch_types=[
        pltpu.VMEM((2048 * 4,), jnp.int32),
        pltpu.VMEM((_RING, _CH, _C), jnp.float32),
        pltpu.VMEM((_NCH, _CH), jnp.int32),
        pltpu.VMEM((_NCH, _CH), jnp.int32),
        pltpu.VMEM((_CH,), jnp.float32),
        pltpu.VMEM((_ZB,), jnp.float32),
        pltpu.SemaphoreType.DMA,
        pltpu.SemaphoreType.DMA,
        [pltpu.SemaphoreType.DMA] * _RING,
        [pltpu.SemaphoreType.DMA] * _RING,
    ],
)
def _sc_scatter(features_hbm, indices_hbm, g_hbm, mask_hbm, *scratch):
    _sc_body(features_hbm, indices_hbm, g_hbm, mask_hbm, *scratch)


def _tc_body(se_ref, so_ref, m0a_ref, m0b_ref, m1a_ref, m1b_ref,
             e_ref, o_ref, o_out_ref):
    se = se_ref[0]
    so = so_ref[0]
    for s in range(_CB // _C):
        m0 = m0a_ref[0, s] + m0b_ref[0, s]
        m1 = m1a_ref[0, s] + m1b_ref[0, s]
        em = e_ref[pl.ds(s * _C, _C), :] * m0[:, None]
        om = o_ref[pl.ds(s * _C, _C), :] * m1[:, None]
        o_out_ref[pl.ds(s * _C, _C), :] = (
            jax.lax.dot(em, se, preferred_element_type=jnp.float32)
            + jax.lax.dot(om, so, preferred_element_type=jnp.float32)
        )


def kernel(features, indices):
    g, mask = _sc_scatter(features, indices.reshape(-1))
    row = jnp.arange(_C)[None, :, None]
    col = jnp.arange(2 * _C)[None, None, :]
    par = jnp.arange(2)[:, None, None]
    sel = (col == 2 * row + par).astype(jnp.float32)
    nblk = _NHW // _CB
    mb = _CB // _C
    zoff = _NHW // _C
    coff = _GROWS // _C
    mask3 = mask.reshape(_MASKLEN // _CB, mb, _C)
    out_p = pl.pallas_call(
        _tc_body,
        grid=(nblk,),
        in_specs=[
            pl.BlockSpec((1, _C, 2 * _C), lambda j: (0, 0, 0)),
            pl.BlockSpec((1, _C, 2 * _C), lambda j: (1, 0, 0)),
            pl.BlockSpec((1, mb, _C), lambda j: (j, 0, 0)),
            pl.BlockSpec((1, mb, _C),
                         lambda j: (coff // mb + j, 0, 0)),
            pl.BlockSpec((1, mb, _C),
                         lambda j: (zoff // mb + j, 0, 0)),
            pl.BlockSpec((1, mb, _C),
                         lambda j: ((coff + zoff) // mb + j, 0, 0)),
            pl.BlockSpec((_CB, _C), lambda j: (j, 0)),
            pl.BlockSpec((_CB, _C), lambda j: (_NHW // _CB + j, 0)),
        ],
        out_specs=pl.BlockSpec((_CB, 2 * _C), lambda j: (j, 0)),
        out_shape=jax.ShapeDtypeStruct((_NHW, 2 * _C), jnp.float32),
    )(sel, sel, mask3, mask3, mask3, mask3, g, g)
    return out_p.reshape(_N, _H, _W, _C * _D).transpose(0, 3, 1, 2)


# speedup vs baseline: 8.9784x; 1.3400x over previous
import functools

import jax
import jax.numpy as jnp
from jax import lax
from jax.experimental import pallas as pl
from jax.experimental.pallas import tpu as pltpu
from jax.experimental.pallas import tpu_sc as plsc

_N, _C, _D, _H, _W = 4, 128, 2, 200, 176
_NNZ = 60000
_NHW = _N * _H * _W
_NWORK = 32
_CH = 128
_NGRP = 469
_NPAD = _NGRP * _CH
_NCH = 15
_RING = 5
_LEAD = 3
_SPARE = 5120
_GROWS = 2 * _NHW + _SPARE
_MASKLEN = 2 * _GROWS
_MZ_PER_TILE = _GROWS // 16
_ZB = 2240
_CB = 2560


def _sc_body(features_hbm, idx4_hbm, g_hbm, mask_hbm,
             bz_v, fbuf, lin_v, mlin_v, ones_v, zbuf,
             zsem, msem, gsems, ssems):
    core = lax.axis_index("c")
    sub = lax.axis_index("s")
    wid = sub * 2 + core
    g0 = wid * 15 - jnp.maximum(wid - 21, 0)
    nch_w = jnp.where(wid < 21, 15, 14)
    v0 = g0 * _CH
    sb = jnp.minimum(g0, _NGRP - _NCH) * _CH

    zeros16 = jnp.zeros((16,), jnp.float32)

    def fill(i, _):
        zbuf[pl.ds(i * 16, 16)] = zeros16
        return 0
    lax.fori_loop(0, _ZB // 16, fill, 0)

    def fill1(i, _):
        ones_v[pl.ds(i * 16, 16)] = zeros16 + 1.0
        return 0
    lax.fori_loop(0, _CH // 16, fill1, 0)

    zd = []
    mbase = core * _GROWS + sub * _MZ_PER_TILE
    for k in range(_MZ_PER_TILE // _ZB):
        zd.append(pltpu.async_copy(
            zbuf, mask_hbm.at[pl.ds(mbase + k * _ZB, _ZB)], zsem))

    nst = _NCH * _CH
    for comp in range(4):
        pltpu.sync_copy(idx4_hbm.at[comp, pl.ds(sb, nst)],
                        bz_v.at[comp, pl.ds(0, nst)])

    def chunk_start(j):
        return jnp.minimum(v0 + jnp.minimum(j, nch_w - 1) * _CH,
                           _NNZ - _CH)

    def chunk(j, _):
        off = chunk_start(j) - sb
        for s in range(_CH // 16):
            o = off + s * 16
            bv = bz_v[0, pl.ds(o, 16)]
            zv = bz_v[1, pl.ds(o, 16)]
            yv = bz_v[2, pl.ds(o, 16)]
            xv = bz_v[3, pl.ds(o, 16)]
            lin = zv * _NHW + (bv * _H + yv) * _W + xv
            lin_v[j, pl.ds(s * 16, 16)] = lin
            mlin_v[j, pl.ds(s * 16, 16)] = core * _GROWS + lin
        return 0

    lax.fori_loop(0, _NCH, chunk, 0)

    for d in zd:
        d.wait()
    plsc.subcore_barrier()

    md = [pltpu.async_copy(ones_v, mask_hbm.at[mlin_v.at[j]], msem)
          for j in range(_NCH)]

    gd = [None] * _NCH
    sd = [None] * _NCH

    def fire_gather(j):
        return pltpu.async_copy(
            features_hbm.at[pl.ds(chunk_start(j), _CH)], fbuf.at[j % _RING],
            gsems[j % _RING])

    def fire_scatter(j):
        return pltpu.async_copy(fbuf.at[j % _RING], g_hbm.at[lin_v.at[j]],
                                ssems[j % _RING])

    for j in range(_NCH + _LEAD):
        if j < _NCH:
            if j >= _RING:
                sd[j - _RING].wait()
            gd[j] = fire_gather(j)
        if j >= _LEAD:
            gd[j - _LEAD].wait()
            sd[j - _LEAD] = fire_scatter(j - _LEAD)
    for j in range(_NCH - _RING, _NCH):
        sd[j].wait()
    for d in md:
        d.wait()


@functools.partial(
    pl.kernel,
    out_type=(
        jax.ShapeDtypeStruct((_GROWS, _C), jnp.float32),
        jax.ShapeDtypeStruct((_MASKLEN,), jnp.float32),
    ),
    mesh=plsc.VectorSubcoreMesh(core_axis_name="c", subcore_axis_name="s"),
    compiler_params=pltpu.CompilerParams(needs_layout_passes=False),
    scratch_types=[
        pltpu.VMEM((4, _NCH * _CH), jnp.int32),
        pltpu.VMEM((_RING, _CH, _C), jnp.float32),
        pltpu.VMEM((_NCH, _CH), jnp.int32),
        pltpu.VMEM((_NCH, _CH), jnp.int32),
        pltpu.VMEM((_CH,), jnp.float32),
        pltpu.VMEM((_ZB,), jnp.float32),
        pltpu.SemaphoreType.DMA,
        pltpu.SemaphoreType.DMA,
        [pltpu.SemaphoreType.DMA] * _RING,
        [pltpu.SemaphoreType.DMA] * _RING,
    ],
)
def _sc_scatter(features_hbm, idx4_hbm, g_hbm, mask_hbm, *scratch):
    _sc_body(features_hbm, idx4_hbm, g_hbm, mask_hbm, *scratch)


def _tc_body(se_ref, so_ref, m0a_ref, m0b_ref, m1a_ref, m1b_ref,
             e_ref, o_ref, o_out_ref):
    se = se_ref[0]
    so = so_ref[0]
    for s in range(_CB // _C):
        m0 = m0a_ref[0, s] + m0b_ref[0, s]
        m1 = m1a_ref[0, s] + m1b_ref[0, s]
        em = e_ref[pl.ds(s * _C, _C), :] * m0[:, None]
        om = o_ref[pl.ds(s * _C, _C), :] * m1[:, None]
        o_out_ref[pl.ds(s * _C, _C), :] = (
            jax.lax.dot(em, se, preferred_element_type=jnp.float32)
            + jax.lax.dot(om, so, preferred_element_type=jnp.float32)
        )


def kernel(features, indices):
    idx4 = jnp.pad(indices, ((0, _NPAD - _NNZ), (0, 0))).T
    g, mask = _sc_scatter(features, idx4)
    row = jnp.arange(_C)[None, :, None]
    col = jnp.arange(2 * _C)[None, None, :]
    par = jnp.arange(2)[:, None, None]
    sel = (col == 2 * row + par).astype(jnp.float32)
    nblk = _NHW // _CB
    mb = _CB // _C
    zoff = _NHW // _C
    coff = _GROWS // _C
    mask3 = mask.reshape(_MASKLEN // _CB, mb, _C)
    out_p = pl.pallas_call(
        _tc_body,
        grid=(nblk,),
        in_specs=[
            pl.BlockSpec((1, _C, 2 * _C), lambda j: (0, 0, 0)),
            pl.BlockSpec((1, _C, 2 * _C), lambda j: (1, 0, 0)),
            pl.BlockSpec((1, mb, _C), lambda j: (j, 0, 0)),
            pl.BlockSpec((1, mb, _C),
                         lambda j: (coff // mb + j, 0, 0)),
            pl.BlockSpec((1, mb, _C),
                         lambda j: (zoff // mb + j, 0, 0)),
            pl.BlockSpec((1, mb, _C),
                         lambda j: ((coff + zoff) // mb + j, 0, 0)),
            pl.BlockSpec((_CB, _C), lambda j: (j, 0)),
            pl.BlockSpec((_CB, _C), lambda j: (_NHW // _CB + j, 0)),
        ],
        out_specs=pl.BlockSpec((_CB, 2 * _C), lambda j: (j, 0)),
        out_shape=jax.ShapeDtypeStruct((_NHW, 2 * _C), jnp.float32),
    )(sel, sel, mask3, mask3, mask3, mask3, g, g)
    return out_p.reshape(_N, _H, _W, _C * _D).transpose(0, 3, 1, 2)
